# baseline (speedup 1.0000x reference)
import functools

import jax
import jax.numpy as jnp
from jax.experimental import pallas as pl
from jax.experimental.pallas import tpu as pltpu

_K = 8
_IBIG = jnp.iinfo(jnp.int32).max


def _knnc_body(dist_ref, labels_ref, out_ref, acc_val, acc_idx, acc_lbl,
               *, n_cols, block_c):
    j = pl.program_id(0)
    nblk = pl.num_programs(0)
    rows = dist_ref.shape[0]

    @pl.when(j == 0)
    def _init():
        acc_val[...] = jnp.full((rows, _K), jnp.inf, jnp.float32)
        acc_idx[...] = jnp.full((rows, _K), _IBIG, jnp.int32)
        acc_lbl[...] = jnp.full((rows, _K), _IBIG, jnp.int32)

    col0 = j * block_c
    colid = col0 + jax.lax.broadcasted_iota(jnp.int32, (rows, block_c), 1)
    d = dist_ref[...]
    d = jnp.where(colid < n_cols, d, jnp.inf)
    lrow = labels_ref[:, pl.ds(col0, block_c)]

    bv, bi, bl = [], [], []
    for _ in range(_K):
        m = jnp.min(d, axis=1)
        is_min = d == m[:, None]
        idx = jnp.min(jnp.where(is_min, colid, _IBIG), axis=1)
        at = colid == idx[:, None]
        lbl = jnp.min(jnp.where(at, lrow, _IBIG), axis=1)
        d = jnp.where(at, jnp.inf, d)
        bv.append(m)
        bi.append(idx)
        bl.append(lbl)

    allv = jnp.concatenate([acc_val[...]] + [v[:, None] for v in bv], axis=1)
    alli = jnp.concatenate([acc_idx[...]] + [v[:, None] for v in bi], axis=1)
    alll = jnp.concatenate([acc_lbl[...]] + [v[:, None] for v in bl], axis=1)

    ov, oi, ol = [], [], []
    for _ in range(_K):
        m = jnp.min(allv, axis=1)
        is_min = allv == m[:, None]
        idx = jnp.min(jnp.where(is_min, alli, _IBIG), axis=1)
        sel = is_min & (alli == idx[:, None])
        lbl = jnp.min(jnp.where(sel, alll, _IBIG), axis=1)
        allv = jnp.where(sel, jnp.inf, allv)
        ov.append(m)
        oi.append(idx)
        ol.append(lbl)
    acc_val[...] = jnp.concatenate([v[:, None] for v in ov], axis=1)
    acc_idx[...] = jnp.concatenate([v[:, None] for v in oi], axis=1)
    acc_lbl[...] = jnp.concatenate([v[:, None] for v in ol], axis=1)

    @pl.when(j == nblk - 1)
    def _vote():
        lab = acc_lbl[...]
        cnt = jnp.ones((rows, _K), jnp.int32)
        for s in range(1, _K):
            rolled = jnp.concatenate([lab[:, s:], lab[:, :s]], axis=1)
            cnt = cnt + (lab == rolled).astype(jnp.int32)
        maxc = jnp.max(cnt, axis=1)
        masked = jnp.where(cnt == maxc[:, None], lab, _IBIG)
        out_ref[...] = jnp.min(masked, axis=1)


@jax.jit
def kernel(distances, labels):
    rows, n_cols = distances.shape
    block_c = 512
    nblk = pl.cdiv(n_cols, block_c)
    n_pad = nblk * block_c
    labels2d = jnp.pad(labels, (0, n_pad - n_cols)).reshape(1, n_pad)
    body = functools.partial(_knnc_body, n_cols=n_cols, block_c=block_c)
    return pl.pallas_call(
        body,
        grid=(nblk,),
        in_specs=[
            pl.BlockSpec((rows, block_c), lambda j: (0, j)),
            pl.BlockSpec((1, n_pad), lambda j: (0, 0)),
        ],
        out_specs=pl.BlockSpec((rows,), lambda j: (0,)),
        out_shape=jax.ShapeDtypeStruct((rows,), jnp.int32),
        scratch_shapes=[
            pltpu.VMEM((rows, _K), jnp.float32),
            pltpu.VMEM((rows, _K), jnp.int32),
            pltpu.VMEM((rows, _K), jnp.int32),
        ],
    )(distances, labels2d)
